# baseline (speedup 1.0000x reference)
import functools

import jax
import jax.numpy as jnp
from jax import lax
from jax.experimental import pallas as pl
from jax.experimental.pallas import tpu as pltpu
from jax.experimental.pallas import tpu_sc as plsc

N_NODES = 10000
D = 128
N_EDGES = 320000
NC = 2
NS = 16
NW = NC * NS
K = 128
CHUNKS = -(-N_EDGES // (NW * K))
EPT = CHUNKS * K
DUMP = N_NODES
ACC_ROWS = N_NODES + 8
STRIPE = 624
TAIL0 = NS * STRIPE
TAIL = N_NODES - TAIL0


def _seg_sum_partials(x, idx, zeros):
    mesh = plsc.VectorSubcoreMesh(core_axis_name="c", subcore_axis_name="s")

    @functools.partial(
        pl.kernel,
        out_type=jax.ShapeDtypeStruct((NC, N_NODES, D), jnp.float32),
        mesh=mesh,
        scratch_types=[
            pltpu.VMEM((2, K), jnp.int32),
            pltpu.VMEM((2, K), jnp.int32),
            pltpu.VMEM((K, D), jnp.float32),
            pltpu.VMEM((K, D), jnp.float32),
            pltpu.VMEM_SHARED((ACC_ROWS, D), jnp.float32),
            pltpu.SemaphoreType.DMA,
            pltpu.SemaphoreType.DMA,
            pltpu.SemaphoreType.DMA,
            pltpu.SemaphoreType.DMA,
        ],
    )
    def body(x_hbm, idx_hbm, zero_hbm, out_hbm,
             ib_a, ib_b, rows_a, rows_b, acc_sh,
             semi_a, semi_b, semr_a, semr_b):
        c = lax.axis_index("c")
        s = lax.axis_index("s")
        wid = s * NC + c
        r0 = s * STRIPE
        pltpu.sync_copy(zero_hbm.at[pl.ds(r0, STRIPE)],
                        acc_sh.at[pl.ds(r0, STRIPE)])

        @pl.when(s == NS - 1)
        def _():
            pltpu.sync_copy(zero_hbm.at[pl.ds(TAIL0, TAIL)],
                            acc_sh.at[pl.ds(TAIL0, TAIL)])

        plsc.subcore_barrier()

        pltpu.async_copy(idx_hbm.at[wid, 0], ib_a, semi_a)
        pltpu.make_async_copy(idx_hbm.at[wid, 0], ib_a, semi_a).wait()
        pltpu.async_copy(x_hbm.at[ib_a.at[0]], rows_a, semr_a)
        pltpu.async_copy(idx_hbm.at[wid, 1], ib_b, semi_b)

        def step(i, carry):
            g = i * 2
            pltpu.make_async_copy(idx_hbm.at[wid, 0], ib_b, semi_b).wait()
            pltpu.async_copy(x_hbm.at[ib_b.at[0]], rows_b, semr_b)
            pltpu.make_async_copy(x_hbm.at[ib_a.at[0]], rows_a, semr_a).wait()
            pltpu.sync_copy(rows_a, acc_sh.at[ib_a.at[1]], add=True)
            pltpu.async_copy(idx_hbm.at[wid, g + 2], ib_a, semi_a)
            pltpu.make_async_copy(x_hbm.at[ib_b.at[0]], rows_b, semr_b).wait()
            pltpu.sync_copy(rows_b, acc_sh.at[ib_b.at[1]], add=True)
            pltpu.make_async_copy(idx_hbm.at[wid, 0], ib_a, semi_a).wait()
            pltpu.async_copy(x_hbm.at[ib_a.at[0]], rows_a, semr_a)
            pltpu.async_copy(idx_hbm.at[wid, g + 3], ib_b, semi_b)
            return carry

        lax.fori_loop(0, (CHUNKS - 1) // 2, step, 0)
        pltpu.make_async_copy(idx_hbm.at[wid, 0], ib_b, semi_b).wait()
        pltpu.make_async_copy(x_hbm.at[ib_a.at[0]], rows_a, semr_a).wait()
        pltpu.sync_copy(rows_a, acc_sh.at[ib_a.at[1]], add=True)
        plsc.subcore_barrier()
        pltpu.sync_copy(acc_sh.at[pl.ds(r0, STRIPE)],
                        out_hbm.at[c, pl.ds(r0, STRIPE)])

        @pl.when(s == NS - 1)
        def _():
            pltpu.sync_copy(acc_sh.at[pl.ds(TAIL0, TAIL)],
                            out_hbm.at[c, pl.ds(TAIL0, TAIL)])

    return body(x, idx, zeros)


def _mlp(x, p0, p1, wa, wb, normalize):
    BR = 1000
    grid = (N_NODES // BR,)

    def body(x_b, p0_b, p1_b, wa_b, wb_b, o_b):
        h = x_b[...] + p0_b[...] + p1_b[...]
        h = jnp.dot(h, wa_b[...], preferred_element_type=jnp.float32)
        h = jnp.maximum(h, 0.0)
        h = jnp.dot(h, wb_b[...], preferred_element_type=jnp.float32)
        if normalize:
            n = jnp.sqrt(jnp.sum(h * h, axis=1, keepdims=True))
            h = h / jnp.maximum(n, 1e-12)
        o_b[...] = h

    return pl.pallas_call(
        body,
        grid=grid,
        in_specs=[
            pl.BlockSpec((BR, D), lambda i: (i, 0)),
            pl.BlockSpec((BR, D), lambda i: (i, 0)),
            pl.BlockSpec((BR, D), lambda i: (i, 0)),
            pl.BlockSpec((D, D), lambda i: (0, 0)),
            pl.BlockSpec((D, D), lambda i: (0, 0)),
        ],
        out_specs=pl.BlockSpec((BR, D), lambda i: (i, 0)),
        out_shape=jax.ShapeDtypeStruct((N_NODES, D), jnp.float32),
    )(x, p0, p1, wa, wb)


def _pack_edges(edge_index):
    ei = edge_index.astype(jnp.int32)
    pad = NW * EPT - N_EDGES
    srcp = jnp.concatenate([ei[0], jnp.zeros((pad,), jnp.int32)])
    dstp = jnp.concatenate([ei[1], jnp.full((pad,), DUMP, jnp.int32)])
    idx = jnp.stack([srcp.reshape(NW, CHUNKS, K),
                     dstp.reshape(NW, CHUNKS, K)], axis=2)
    extra = jnp.zeros((NW, 1, 2, K), jnp.int32)
    return jnp.concatenate([idx, extra], axis=1)


def kernel(x, edge_index, w1a, w1b, w2a, w2b):
    idx = _pack_edges(edge_index)
    zeros = jnp.zeros((N_NODES, D), jnp.float32)

    p = _seg_sum_partials(x, idx, zeros)
    h1 = _mlp(x, p[0], p[1], w1a, w1b, normalize=False)
    q = _seg_sum_partials(h1, idx, zeros)
    return _mlp(h1, q[0], q[1], w2a, w2b, normalize=True)


# speedup vs baseline: 1.1057x; 1.1057x over previous
import functools

import jax
import jax.numpy as jnp
from jax import lax
from jax.experimental import pallas as pl
from jax.experimental.pallas import tpu as pltpu
from jax.experimental.pallas import tpu_sc as plsc

N_NODES = 10000
D = 128
N_EDGES = 320000
NC = 2
NS = 16
NW = NC * NS
K = 128
CHUNKS = -(-N_EDGES // (NW * K))
EPT = CHUNKS * K
EPT_REAL = N_EDGES // NW
PAD = EPT - EPT_REAL
ACC_ROWS = N_NODES + PAD
STRIPE = 624
TAIL0 = NS * STRIPE
TAIL = N_NODES - TAIL0


def _seg_sum_partials(x, idx, zeros):
    mesh = plsc.VectorSubcoreMesh(core_axis_name="c", subcore_axis_name="s")

    @functools.partial(
        pl.kernel,
        out_type=jax.ShapeDtypeStruct((NC, N_NODES, D), jnp.float32),
        mesh=mesh,
        scratch_types=[
            pltpu.VMEM((2, K), jnp.int32),
            pltpu.VMEM((2, K), jnp.int32),
            pltpu.VMEM((K, D), jnp.float32),
            pltpu.VMEM((K, D), jnp.float32),
            pltpu.VMEM_SHARED((ACC_ROWS, D), jnp.float32),
            pltpu.SemaphoreType.DMA,
            pltpu.SemaphoreType.DMA,
            pltpu.SemaphoreType.DMA,
            pltpu.SemaphoreType.DMA,
        ],
    )
    def body(x_hbm, idx_hbm, zero_hbm, out_hbm,
             ib_a, ib_b, rows_a, rows_b, acc_sh,
             semi_a, semi_b, semr_a, semr_b):
        c = lax.axis_index("c")
        s = lax.axis_index("s")
        wid = s * NC + c
        r0 = s * STRIPE
        pltpu.sync_copy(zero_hbm.at[pl.ds(r0, STRIPE)],
                        acc_sh.at[pl.ds(r0, STRIPE)])

        @pl.when(s == NS - 1)
        def _():
            pltpu.sync_copy(zero_hbm.at[pl.ds(TAIL0, TAIL)],
                            acc_sh.at[pl.ds(TAIL0, TAIL)])

        plsc.subcore_barrier()

        pltpu.async_copy(idx_hbm.at[wid, 0], ib_a, semi_a)
        pltpu.make_async_copy(idx_hbm.at[wid, 0], ib_a, semi_a).wait()
        pltpu.async_copy(x_hbm.at[ib_a.at[0]], rows_a, semr_a)
        pltpu.async_copy(idx_hbm.at[wid, 1], ib_b, semi_b)

        def step(i, carry):
            g = i * 2
            pltpu.make_async_copy(idx_hbm.at[wid, 0], ib_b, semi_b).wait()
            pltpu.async_copy(x_hbm.at[ib_b.at[0]], rows_b, semr_b)
            pltpu.make_async_copy(x_hbm.at[ib_a.at[0]], rows_a, semr_a).wait()
            pltpu.sync_copy(rows_a, acc_sh.at[ib_a.at[1]], add=True)
            pltpu.async_copy(idx_hbm.at[wid, g + 2], ib_a, semi_a)
            pltpu.make_async_copy(x_hbm.at[ib_b.at[0]], rows_b, semr_b).wait()
            pltpu.sync_copy(rows_b, acc_sh.at[ib_b.at[1]], add=True)
            pltpu.make_async_copy(idx_hbm.at[wid, 0], ib_a, semi_a).wait()
            pltpu.async_copy(x_hbm.at[ib_a.at[0]], rows_a, semr_a)
            pltpu.async_copy(idx_hbm.at[wid, g + 3], ib_b, semi_b)
            return carry

        lax.fori_loop(0, (CHUNKS - 1) // 2, step, 0)
        pltpu.make_async_copy(idx_hbm.at[wid, 0], ib_b, semi_b).wait()
        pltpu.make_async_copy(x_hbm.at[ib_a.at[0]], rows_a, semr_a).wait()
        pltpu.sync_copy(rows_a, acc_sh.at[ib_a.at[1]], add=True)
        plsc.subcore_barrier()
        pltpu.sync_copy(acc_sh.at[pl.ds(r0, STRIPE)],
                        out_hbm.at[c, pl.ds(r0, STRIPE)])

        @pl.when(s == NS - 1)
        def _():
            pltpu.sync_copy(acc_sh.at[pl.ds(TAIL0, TAIL)],
                            out_hbm.at[c, pl.ds(TAIL0, TAIL)])

    return body(x, idx, zeros)


def _mlp(x, p0, p1, wa, wb, normalize):
    BR = 1000
    grid = (N_NODES // BR,)

    def body(x_b, p0_b, p1_b, wa_b, wb_b, o_b):
        h = x_b[...] + p0_b[...] + p1_b[...]
        h = jnp.dot(h, wa_b[...], preferred_element_type=jnp.float32)
        h = jnp.maximum(h, 0.0)
        h = jnp.dot(h, wb_b[...], preferred_element_type=jnp.float32)
        if normalize:
            n = jnp.sqrt(jnp.sum(h * h, axis=1, keepdims=True))
            h = h / jnp.maximum(n, 1e-12)
        o_b[...] = h

    return pl.pallas_call(
        body,
        grid=grid,
        in_specs=[
            pl.BlockSpec((BR, D), lambda i: (i, 0)),
            pl.BlockSpec((BR, D), lambda i: (i, 0)),
            pl.BlockSpec((BR, D), lambda i: (i, 0)),
            pl.BlockSpec((D, D), lambda i: (0, 0)),
            pl.BlockSpec((D, D), lambda i: (0, 0)),
        ],
        out_specs=pl.BlockSpec((BR, D), lambda i: (i, 0)),
        out_shape=jax.ShapeDtypeStruct((N_NODES, D), jnp.float32),
    )(x, p0, p1, wa, wb)


def _pack_edges(edge_index):
    ei = edge_index.astype(jnp.int32)
    srcp = jnp.pad(ei[0].reshape(NW, EPT_REAL), ((0, 0), (0, PAD)))
    pad_dst = jnp.broadcast_to(
        N_NODES + jnp.arange(PAD, dtype=jnp.int32), (NW, PAD))
    dstp = jnp.concatenate([ei[1].reshape(NW, EPT_REAL), pad_dst], axis=1)
    idx = jnp.stack([srcp.reshape(NW, CHUNKS, K),
                     dstp.reshape(NW, CHUNKS, K)], axis=2)
    extra = jnp.zeros((NW, 1, 2, K), jnp.int32)
    return jnp.concatenate([idx, extra], axis=1)


def kernel(x, edge_index, w1a, w1b, w2a, w2b):
    idx = _pack_edges(edge_index)
    zeros = jnp.zeros((N_NODES, D), jnp.float32)

    p = _seg_sum_partials(x, idx, zeros)
    h1 = _mlp(x, p[0], p[1], w1a, w1b, normalize=False)
    q = _seg_sum_partials(h1, idx, zeros)
    return _mlp(h1, q[0], q[1], w2a, w2b, normalize=True)


# speedup vs baseline: 1.7476x; 1.5805x over previous
import functools

import jax
import jax.numpy as jnp
from jax import lax
from jax.experimental import pallas as pl
from jax.experimental.pallas import tpu as pltpu
from jax.experimental.pallas import tpu_sc as plsc

N_NODES = 10000
D = 128
N_EDGES = 320000
NC = 2
NS = 16
NW = NC * NS
K = 80
EPT = N_EDGES // NW
CHUNKS = EPT // K
ACC_ROWS = N_NODES
STRIPE = 624
TAIL0 = NS * STRIPE
TAIL = N_NODES - TAIL0


def _seg_sum_partials(x, idx, zeros):
    mesh = plsc.VectorSubcoreMesh(core_axis_name="c", subcore_axis_name="s")

    @functools.partial(
        pl.kernel,
        out_type=jax.ShapeDtypeStruct((NC, N_NODES, D), jnp.float32),
        mesh=mesh,
        scratch_types=[
            pltpu.VMEM((2, K), jnp.int32),
            pltpu.VMEM((2, K), jnp.int32),
            pltpu.VMEM((K, D), jnp.float32),
            pltpu.VMEM((K, D), jnp.float32),
            pltpu.VMEM_SHARED((ACC_ROWS, D), jnp.float32),
            pltpu.SemaphoreType.DMA,
            pltpu.SemaphoreType.DMA,
            pltpu.SemaphoreType.DMA,
            pltpu.SemaphoreType.DMA,
        ],
    )
    def body(x_hbm, idx_hbm, zero_hbm, out_hbm,
             ib_a, ib_b, rows_a, rows_b, acc_sh,
             semi_a, semi_b, semr_a, semr_b):
        c = lax.axis_index("c")
        s = lax.axis_index("s")
        wid = s * NC + c
        r0 = s * STRIPE
        pltpu.sync_copy(zero_hbm.at[pl.ds(r0, STRIPE)],
                        acc_sh.at[pl.ds(r0, STRIPE)])

        @pl.when(s == NS - 1)
        def _():
            pltpu.sync_copy(zero_hbm.at[pl.ds(TAIL0, TAIL)],
                            acc_sh.at[pl.ds(TAIL0, TAIL)])

        plsc.subcore_barrier()

        pltpu.async_copy(idx_hbm.at[wid, 0], ib_a, semi_a)
        pltpu.make_async_copy(idx_hbm.at[wid, 0], ib_a, semi_a).wait()
        pltpu.async_copy(x_hbm.at[ib_a.at[0]], rows_a, semr_a)
        pltpu.async_copy(idx_hbm.at[wid, 1], ib_b, semi_b)

        def step(i, carry):
            g = i * 2
            pltpu.make_async_copy(idx_hbm.at[wid, 0], ib_b, semi_b).wait()
            pltpu.async_copy(x_hbm.at[ib_b.at[0]], rows_b, semr_b)
            pltpu.make_async_copy(x_hbm.at[ib_a.at[0]], rows_a, semr_a).wait()
            pltpu.sync_copy(rows_a, acc_sh.at[ib_a.at[1]], add=True)
            pltpu.async_copy(idx_hbm.at[wid, g + 2], ib_a, semi_a)
            pltpu.make_async_copy(x_hbm.at[ib_b.at[0]], rows_b, semr_b).wait()
            pltpu.sync_copy(rows_b, acc_sh.at[ib_b.at[1]], add=True)
            pltpu.make_async_copy(idx_hbm.at[wid, 0], ib_a, semi_a).wait()
            pltpu.async_copy(x_hbm.at[ib_a.at[0]], rows_a, semr_a)
            pltpu.async_copy(idx_hbm.at[wid, g + 3], ib_b, semi_b)
            return carry

        lax.fori_loop(0, (CHUNKS - 1) // 2, step, 0)
        pltpu.make_async_copy(idx_hbm.at[wid, 0], ib_b, semi_b).wait()
        pltpu.make_async_copy(x_hbm.at[ib_a.at[0]], rows_a, semr_a).wait()
        pltpu.sync_copy(rows_a, acc_sh.at[ib_a.at[1]], add=True)
        plsc.subcore_barrier()
        pltpu.sync_copy(acc_sh.at[pl.ds(r0, STRIPE)],
                        out_hbm.at[c, pl.ds(r0, STRIPE)])

        @pl.when(s == NS - 1)
        def _():
            pltpu.sync_copy(acc_sh.at[pl.ds(TAIL0, TAIL)],
                            out_hbm.at[c, pl.ds(TAIL0, TAIL)])

    return body(x, idx, zeros)


def _mlp(x, p0, p1, wa, wb, normalize):
    BR = 1000
    grid = (N_NODES // BR,)

    def body(x_b, p0_b, p1_b, wa_b, wb_b, o_b):
        h = x_b[...] + p0_b[...] + p1_b[...]
        h = jnp.dot(h, wa_b[...], preferred_element_type=jnp.float32)
        h = jnp.maximum(h, 0.0)
        h = jnp.dot(h, wb_b[...], preferred_element_type=jnp.float32)
        if normalize:
            n = jnp.sqrt(jnp.sum(h * h, axis=1, keepdims=True))
            h = h / jnp.maximum(n, 1e-12)
        o_b[...] = h

    return pl.pallas_call(
        body,
        grid=grid,
        in_specs=[
            pl.BlockSpec((BR, D), lambda i: (i, 0)),
            pl.BlockSpec((BR, D), lambda i: (i, 0)),
            pl.BlockSpec((BR, D), lambda i: (i, 0)),
            pl.BlockSpec((D, D), lambda i: (0, 0)),
            pl.BlockSpec((D, D), lambda i: (0, 0)),
        ],
        out_specs=pl.BlockSpec((BR, D), lambda i: (i, 0)),
        out_shape=jax.ShapeDtypeStruct((N_NODES, D), jnp.float32),
    )(x, p0, p1, wa, wb)


def _pack_edges(edge_index):
    ei = edge_index.astype(jnp.int32)
    idx = jnp.stack([ei[0].reshape(NW, CHUNKS, K),
                     ei[1].reshape(NW, CHUNKS, K)], axis=2)
    extra = jnp.zeros((NW, 1, 2, K), jnp.int32)
    return jnp.concatenate([idx, extra], axis=1)


def kernel(x, edge_index, w1a, w1b, w2a, w2b):
    idx = _pack_edges(edge_index)
    zeros = jnp.zeros((N_NODES, D), jnp.float32)

    p = _seg_sum_partials(x, idx, zeros)
    h1 = _mlp(x, p[0], p[1], w1a, w1b, normalize=False)
    q = _seg_sum_partials(h1, idx, zeros)
    return _mlp(h1, q[0], q[1], w2a, w2b, normalize=True)


# speedup vs baseline: 2.0753x; 1.1875x over previous
import functools

import jax
import jax.numpy as jnp
from jax import lax
from jax.experimental import pallas as pl
from jax.experimental.pallas import tpu as pltpu
from jax.experimental.pallas import tpu_sc as plsc

N_NODES = 10000
D = 128
N_EDGES = 320000
NC = 2
NS = 16
NW = NC * NS
K = 80
EPT = N_EDGES // NW
CHUNKS = EPT // K
ACC_ROWS = N_NODES
STRIPE = 624
TAIL0 = NS * STRIPE
TAIL = N_NODES - TAIL0


def _seg_sum_partials(x, idx, zeros):
    mesh = plsc.VectorSubcoreMesh(core_axis_name="c", subcore_axis_name="s")

    @functools.partial(
        pl.kernel,
        out_type=jax.ShapeDtypeStruct((NC, N_NODES, D), jnp.float32),
        mesh=mesh,
        scratch_types=[
            pltpu.VMEM((2, K), jnp.int32),
            pltpu.VMEM((2, K), jnp.int32),
            pltpu.VMEM((2, K), jnp.int32),
            pltpu.VMEM((2, K), jnp.int32),
            pltpu.VMEM((K, D), jnp.float32),
            pltpu.VMEM((K, D), jnp.float32),
            pltpu.VMEM_SHARED((ACC_ROWS, D), jnp.float32),
            pltpu.SemaphoreType.DMA,
            pltpu.SemaphoreType.DMA,
            pltpu.SemaphoreType.DMA,
            pltpu.SemaphoreType.DMA,
            pltpu.SemaphoreType.DMA,
            pltpu.SemaphoreType.DMA,
        ],
    )
    def body(x_hbm, idx_hbm, zero_hbm, out_hbm,
             ib0, ib1, ib2, ib3, rows_a, rows_b, acc_sh,
             semi0, semi1, semi2, semi3, semr_a, semr_b):
        c = lax.axis_index("c")
        s = lax.axis_index("s")
        wid = s * NC + c
        r0 = s * STRIPE
        pltpu.sync_copy(zero_hbm.at[pl.ds(r0, STRIPE)],
                        acc_sh.at[pl.ds(r0, STRIPE)])

        @pl.when(s == NS - 1)
        def _():
            pltpu.sync_copy(zero_hbm.at[pl.ds(TAIL0, TAIL)],
                            acc_sh.at[pl.ds(TAIL0, TAIL)])

        plsc.subcore_barrier()

        def widx(n):
            pltpu.make_async_copy(idx_hbm.at[wid, 0], n[0], n[1]).wait()

        def wrows(buf, sem):
            pltpu.make_async_copy(x_hbm.at[ib0.at[0]], buf, sem).wait()

        pltpu.async_copy(idx_hbm.at[wid, 0], ib0, semi0)
        pltpu.async_copy(idx_hbm.at[wid, 1], ib1, semi1)
        pltpu.async_copy(idx_hbm.at[wid, 2], ib2, semi2)
        pltpu.async_copy(idx_hbm.at[wid, 3], ib3, semi3)
        widx((ib0, semi0))
        widx((ib1, semi1))
        pltpu.async_copy(x_hbm.at[ib0.at[0]], rows_a, semr_a)

        def step(i, carry):
            g = i * 4
            pltpu.async_copy(x_hbm.at[ib1.at[0]], rows_b, semr_b)
            wrows(rows_a, semr_a)
            pltpu.sync_copy(rows_a, acc_sh.at[ib0.at[1]], add=True)
            pltpu.async_copy(idx_hbm.at[wid, g + 4], ib0, semi0)
            widx((ib2, semi2))
            pltpu.async_copy(x_hbm.at[ib2.at[0]], rows_a, semr_a)
            wrows(rows_b, semr_b)
            pltpu.sync_copy(rows_b, acc_sh.at[ib1.at[1]], add=True)
            pltpu.async_copy(idx_hbm.at[wid, g + 5], ib1, semi1)
            widx((ib3, semi3))
            pltpu.async_copy(x_hbm.at[ib3.at[0]], rows_b, semr_b)
            wrows(rows_a, semr_a)
            pltpu.sync_copy(rows_a, acc_sh.at[ib2.at[1]], add=True)
            pltpu.async_copy(idx_hbm.at[wid, g + 6], ib2, semi2)
            widx((ib0, semi0))
            pltpu.async_copy(x_hbm.at[ib0.at[0]], rows_a, semr_a)
            wrows(rows_b, semr_b)
            pltpu.sync_copy(rows_b, acc_sh.at[ib3.at[1]], add=True)
            pltpu.async_copy(idx_hbm.at[wid, g + 7], ib3, semi3)
            widx((ib1, semi1))
            return carry

        lax.fori_loop(0, (CHUNKS - 1) // 4, step, 0)
        wrows(rows_a, semr_a)
        pltpu.sync_copy(rows_a, acc_sh.at[ib0.at[1]], add=True)
        widx((ib2, semi2))
        widx((ib3, semi3))
        plsc.subcore_barrier()
        pltpu.sync_copy(acc_sh.at[pl.ds(r0, STRIPE)],
                        out_hbm.at[c, pl.ds(r0, STRIPE)])

        @pl.when(s == NS - 1)
        def _():
            pltpu.sync_copy(acc_sh.at[pl.ds(TAIL0, TAIL)],
                            out_hbm.at[c, pl.ds(TAIL0, TAIL)])

    return body(x, idx, zeros)


def _mlp(x, p0, p1, wa, wb, normalize):
    BR = 1000
    grid = (N_NODES // BR,)

    def body(x_b, p0_b, p1_b, wa_b, wb_b, o_b):
        h = x_b[...] + p0_b[...] + p1_b[...]
        h = jnp.dot(h, wa_b[...], preferred_element_type=jnp.float32)
        h = jnp.maximum(h, 0.0)
        h = jnp.dot(h, wb_b[...], preferred_element_type=jnp.float32)
        if normalize:
            n = jnp.sqrt(jnp.sum(h * h, axis=1, keepdims=True))
            h = h / jnp.maximum(n, 1e-12)
        o_b[...] = h

    return pl.pallas_call(
        body,
        grid=grid,
        in_specs=[
            pl.BlockSpec((BR, D), lambda i: (i, 0)),
            pl.BlockSpec((BR, D), lambda i: (i, 0)),
            pl.BlockSpec((BR, D), lambda i: (i, 0)),
            pl.BlockSpec((D, D), lambda i: (0, 0)),
            pl.BlockSpec((D, D), lambda i: (0, 0)),
        ],
        out_specs=pl.BlockSpec((BR, D), lambda i: (i, 0)),
        out_shape=jax.ShapeDtypeStruct((N_NODES, D), jnp.float32),
    )(x, p0, p1, wa, wb)


def _pack_edges(edge_index):
    ei = edge_index.astype(jnp.int32)
    idx = jnp.stack([ei[0].reshape(NW, CHUNKS, K),
                     ei[1].reshape(NW, CHUNKS, K)], axis=2)
    extra = jnp.zeros((NW, 3, 2, K), jnp.int32)
    return jnp.concatenate([idx, extra], axis=1)


def kernel(x, edge_index, w1a, w1b, w2a, w2b):
    idx = _pack_edges(edge_index)
    zeros = jnp.zeros((N_NODES, D), jnp.float32)

    p = _seg_sum_partials(x, idx, zeros)
    h1 = _mlp(x, p[0], p[1], w1a, w1b, normalize=False)
    q = _seg_sum_partials(h1, idx, zeros)
    return _mlp(h1, q[0], q[1], w2a, w2b, normalize=True)


# speedup vs baseline: 2.2366x; 1.0777x over previous
import functools

import jax
import jax.numpy as jnp
from jax import lax
from jax.experimental import pallas as pl
from jax.experimental.pallas import tpu as pltpu
from jax.experimental.pallas import tpu_sc as plsc

N_NODES = 10000
D = 128
N_EDGES = 320000
NC = 2
NS = 16
NW = NC * NS
K = 80
EPT = N_EDGES // NW
CHUNKS = EPT // K
ACC_ROWS = N_NODES
STRIPE = 624
TAIL0 = NS * STRIPE
TAIL = N_NODES - TAIL0


def _seg_sum_partials(x, idx, zeros):
    mesh = plsc.VectorSubcoreMesh(core_axis_name="c", subcore_axis_name="s")

    @functools.partial(
        pl.kernel,
        out_type=jax.ShapeDtypeStruct((NC, N_NODES, D), jnp.float32),
        mesh=mesh,
        scratch_types=[
            pltpu.VMEM((2, K), jnp.int32),
            pltpu.VMEM((2, K), jnp.int32),
            pltpu.VMEM((2, K), jnp.int32),
            pltpu.VMEM((2, K), jnp.int32),
            pltpu.VMEM((K, D), jnp.float32),
            pltpu.VMEM((K, D), jnp.float32),
            pltpu.VMEM_SHARED((ACC_ROWS, D), jnp.float32),
            pltpu.SemaphoreType.DMA,
            pltpu.SemaphoreType.DMA,
            pltpu.SemaphoreType.DMA,
            pltpu.SemaphoreType.DMA,
            pltpu.SemaphoreType.DMA,
            pltpu.SemaphoreType.DMA,
        ],
    )
    def body(x_hbm, idx_hbm, zero_hbm, out_hbm,
             ib0, ib1, ib2, ib3, rows_a, rows_b, acc_sh,
             semi0, semi1, semi2, semi3, semr_a, semr_b):
        c = lax.axis_index("c")
        s = lax.axis_index("s")
        wid = s * NC + c

        def widx(n):
            pltpu.make_async_copy(idx_hbm.at[wid, 0], n[0], n[1]).wait()

        def wrows(buf, sem):
            pltpu.make_async_copy(x_hbm.at[ib0.at[0]], buf, sem).wait()

        pltpu.async_copy(idx_hbm.at[wid, 0], ib0, semi0)
        pltpu.async_copy(idx_hbm.at[wid, 1], ib1, semi1)
        pltpu.async_copy(idx_hbm.at[wid, 2], ib2, semi2)
        pltpu.async_copy(idx_hbm.at[wid, 3], ib3, semi3)

        r0 = s * STRIPE

        @pl.when(c == 0)
        def _():
            pltpu.sync_copy(x_hbm.at[pl.ds(r0, STRIPE)],
                            acc_sh.at[pl.ds(r0, STRIPE)])

            @pl.when(s == NS - 1)
            def _():
                pltpu.sync_copy(x_hbm.at[pl.ds(TAIL0, TAIL)],
                                acc_sh.at[pl.ds(TAIL0, TAIL)])

        @pl.when(c != 0)
        def _():
            pltpu.sync_copy(zero_hbm.at[pl.ds(r0, STRIPE)],
                            acc_sh.at[pl.ds(r0, STRIPE)])

            @pl.when(s == NS - 1)
            def _():
                pltpu.sync_copy(zero_hbm.at[pl.ds(TAIL0, TAIL)],
                                acc_sh.at[pl.ds(TAIL0, TAIL)])

        plsc.subcore_barrier()
        widx((ib0, semi0))
        widx((ib1, semi1))
        pltpu.async_copy(x_hbm.at[ib0.at[0]], rows_a, semr_a)

        def step(i, carry):
            g = i * 4
            pltpu.async_copy(x_hbm.at[ib1.at[0]], rows_b, semr_b)
            wrows(rows_a, semr_a)
            pltpu.sync_copy(rows_a, acc_sh.at[ib0.at[1]], add=True)
            pltpu.async_copy(idx_hbm.at[wid, g + 4], ib0, semi0)
            widx((ib2, semi2))
            pltpu.async_copy(x_hbm.at[ib2.at[0]], rows_a, semr_a)
            wrows(rows_b, semr_b)
            pltpu.sync_copy(rows_b, acc_sh.at[ib1.at[1]], add=True)
            pltpu.async_copy(idx_hbm.at[wid, g + 5], ib1, semi1)
            widx((ib3, semi3))
            pltpu.async_copy(x_hbm.at[ib3.at[0]], rows_b, semr_b)
            wrows(rows_a, semr_a)
            pltpu.sync_copy(rows_a, acc_sh.at[ib2.at[1]], add=True)
            pltpu.async_copy(idx_hbm.at[wid, g + 6], ib2, semi2)
            widx((ib0, semi0))
            pltpu.async_copy(x_hbm.at[ib0.at[0]], rows_a, semr_a)
            wrows(rows_b, semr_b)
            pltpu.sync_copy(rows_b, acc_sh.at[ib3.at[1]], add=True)
            pltpu.async_copy(idx_hbm.at[wid, g + 7], ib3, semi3)
            widx((ib1, semi1))
            return carry

        lax.fori_loop(0, (CHUNKS - 1) // 4, step, 0)
        wrows(rows_a, semr_a)
        pltpu.sync_copy(rows_a, acc_sh.at[ib0.at[1]], add=True)
        widx((ib2, semi2))
        widx((ib3, semi3))
        plsc.subcore_barrier()
        pltpu.sync_copy(acc_sh.at[pl.ds(r0, STRIPE)],
                        out_hbm.at[c, pl.ds(r0, STRIPE)])

        @pl.when(s == NS - 1)
        def _():
            pltpu.sync_copy(acc_sh.at[pl.ds(TAIL0, TAIL)],
                            out_hbm.at[c, pl.ds(TAIL0, TAIL)])

    return body(x, idx, zeros)


def _mlp(p, wa, wb, normalize):
    BR = 2000
    grid = (N_NODES // BR,)

    def body(p_b, wa_b, wb_b, o_b):
        h = p_b[0] + p_b[1]
        h = jnp.dot(h, wa_b[...], preferred_element_type=jnp.float32)
        h = jnp.maximum(h, 0.0)
        h = jnp.dot(h, wb_b[...], preferred_element_type=jnp.float32)
        if normalize:
            n = jnp.sqrt(jnp.sum(h * h, axis=1, keepdims=True))
            h = h / jnp.maximum(n, 1e-12)
        o_b[...] = h

    return pl.pallas_call(
        body,
        grid=grid,
        in_specs=[
            pl.BlockSpec((2, BR, D), lambda i: (0, i, 0)),
            pl.BlockSpec((D, D), lambda i: (0, 0)),
            pl.BlockSpec((D, D), lambda i: (0, 0)),
        ],
        out_specs=pl.BlockSpec((BR, D), lambda i: (i, 0)),
        out_shape=jax.ShapeDtypeStruct((N_NODES, D), jnp.float32),
    )(p, wa, wb)


def _pack_edges(edge_index):
    ei = edge_index.astype(jnp.int32)
    idx = jnp.stack([ei[0].reshape(NW, CHUNKS, K),
                     ei[1].reshape(NW, CHUNKS, K)], axis=2)
    extra = jnp.zeros((NW, 3, 2, K), jnp.int32)
    return jnp.concatenate([idx, extra], axis=1)


def kernel(x, edge_index, w1a, w1b, w2a, w2b):
    idx = _pack_edges(edge_index)
    zeros = jnp.zeros((N_NODES, D), jnp.float32)

    p = _seg_sum_partials(x, idx, zeros)
    h1 = _mlp(p, w1a, w1b, normalize=False)
    q = _seg_sum_partials(h1, idx, zeros)
    return _mlp(q, w2a, w2b, normalize=True)


# speedup vs baseline: 2.2514x; 1.0066x over previous
import functools

import jax
import jax.numpy as jnp
from jax import lax
from jax.experimental import pallas as pl
from jax.experimental.pallas import tpu as pltpu
from jax.experimental.pallas import tpu_sc as plsc

N_NODES = 10000
D = 128
N_EDGES = 320000
NC = 2
NS = 16
NW = NC * NS
K = 80
EPT = N_EDGES // NW
CHUNKS = EPT // K
ACC_ROWS = N_NODES
STRIPE = 624
TAIL0 = NS * STRIPE
TAIL = N_NODES - TAIL0


def _seg_sum_partials(x, idx, zeros):
    mesh = plsc.VectorSubcoreMesh(core_axis_name="c", subcore_axis_name="s")

    @functools.partial(
        pl.kernel,
        out_type=jax.ShapeDtypeStruct((NC, N_NODES, D), jnp.float32),
        mesh=mesh,
        scratch_types=[
            pltpu.VMEM((2, K), jnp.int32),
            pltpu.VMEM((2, K), jnp.int32),
            pltpu.VMEM((2, K), jnp.int32),
            pltpu.VMEM((2, K), jnp.int32),
            pltpu.VMEM((K, D), jnp.float32),
            pltpu.VMEM((K, D), jnp.float32),
            pltpu.VMEM_SHARED((ACC_ROWS, D), jnp.float32),
            pltpu.SemaphoreType.DMA,
            pltpu.SemaphoreType.DMA,
            pltpu.SemaphoreType.DMA,
            pltpu.SemaphoreType.DMA,
            pltpu.SemaphoreType.DMA,
            pltpu.SemaphoreType.DMA,
            pltpu.SemaphoreType.DMA,
            pltpu.SemaphoreType.DMA,
            pltpu.SemaphoreType.DMA,
        ],
    )
    def body(x_hbm, idx_hbm, zero_hbm, out_hbm,
             ib0, ib1, ib2, ib3, rows_a, rows_b, acc_sh,
             semi0, semi1, semi2, semi3, semr_a, semr_b,
             sems_a, sems_b, sem_init):
        c = lax.axis_index("c")
        s = lax.axis_index("s")
        wid = s * NC + c

        ibs = (ib0, ib1, ib2, ib3)
        semis = (semi0, semi1, semi2, semi3)
        rows = (rows_a, rows_b)
        semrs = (semr_a, semr_b)
        semss = (sems_a, sems_b)

        def widx(m4):
            pltpu.make_async_copy(idx_hbm.at[wid, 0], ibs[m4], semis[m4]).wait()

        def wrows(m2):
            pltpu.make_async_copy(x_hbm.at[ib0.at[0]], rows[m2], semrs[m2]).wait()

        def wscat(m2):
            pltpu.make_async_copy(rows[m2], acc_sh.at[ib0.at[1]],
                                  semss[m2]).wait()

        def gather(m4, m2):
            pltpu.async_copy(x_hbm.at[ibs[m4].at[0]], rows[m2], semrs[m2])

        def scatter(m4, m2):
            pltpu.async_copy(rows[m2], acc_sh.at[ibs[m4].at[1]],
                             semss[m2], add=True)

        def refill(jv, m4):
            pltpu.async_copy(idx_hbm.at[wid, jv], ibs[m4], semis[m4])

        for j in range(3):
            refill(j, j)

        r0 = s * STRIPE

        @pl.when(c == 0)
        def _():
            pltpu.async_copy(x_hbm.at[pl.ds(r0, STRIPE)],
                             acc_sh.at[pl.ds(r0, STRIPE)], sem_init)

            @pl.when(s == NS - 1)
            def _():
                pltpu.async_copy(x_hbm.at[pl.ds(TAIL0, TAIL)],
                                 acc_sh.at[pl.ds(TAIL0, TAIL)], sem_init)

        @pl.when(c != 0)
        def _():
            pltpu.async_copy(zero_hbm.at[pl.ds(r0, STRIPE)],
                             acc_sh.at[pl.ds(r0, STRIPE)], sem_init)

            @pl.when(s == NS - 1)
            def _():
                pltpu.async_copy(zero_hbm.at[pl.ds(TAIL0, TAIL)],
                                 acc_sh.at[pl.ds(TAIL0, TAIL)], sem_init)

        widx(0)
        gather(0, 0)
        pltpu.make_async_copy(zero_hbm.at[pl.ds(r0, STRIPE)],
                              acc_sh.at[pl.ds(r0, STRIPE)], sem_init).wait()

        @pl.when(s == NS - 1)
        def _():
            pltpu.make_async_copy(zero_hbm.at[pl.ds(TAIL0, TAIL)],
                                  acc_sh.at[pl.ds(TAIL0, TAIL)],
                                  sem_init).wait()

        plsc.subcore_barrier()

        def slot(jv, jm4, head=False, refill_on=True, gather_on=True):
            if not head:
                wscat((jm4 + 1) % 2)
            if refill_on:
                refill(jv + 3, (jm4 + 3) % 4)
            if gather_on:
                widx((jm4 + 1) % 4)
                gather((jm4 + 1) % 4, (jm4 + 1) % 2)
            wrows(jm4 % 2)
            scatter(jm4 % 4, jm4 % 2)

        slot(0, 0, head=True)
        slot(1, 1)

        def step(i, carry):
            g = i * 4 + 2
            slot(g, 2)
            slot(g + 1, 3)
            slot(g + 2, 0)
            slot(g + 3, 1)
            return carry

        lax.fori_loop(0, (CHUNKS - 5) // 4, step, 0)
        slot(CHUNKS - 3, 2, refill_on=False)
        slot(CHUNKS - 2, 3, refill_on=False)
        slot(CHUNKS - 1, 0, refill_on=False, gather_on=False)
        wscat(0)
        plsc.subcore_barrier()
        pltpu.sync_copy(acc_sh.at[pl.ds(r0, STRIPE)],
                        out_hbm.at[c, pl.ds(r0, STRIPE)])

        @pl.when(s == NS - 1)
        def _():
            pltpu.sync_copy(acc_sh.at[pl.ds(TAIL0, TAIL)],
                            out_hbm.at[c, pl.ds(TAIL0, TAIL)])

    return body(x, idx, zeros)


def _mlp(p, wa, wb, normalize):
    BR = 2000
    grid = (N_NODES // BR,)

    def body(p_b, wa_b, wb_b, o_b):
        h = p_b[0] + p_b[1]
        h = jnp.dot(h, wa_b[...], preferred_element_type=jnp.float32)
        h = jnp.maximum(h, 0.0)
        h = jnp.dot(h, wb_b[...], preferred_element_type=jnp.float32)
        if normalize:
            n = jnp.sqrt(jnp.sum(h * h, axis=1, keepdims=True))
            h = h / jnp.maximum(n, 1e-12)
        o_b[...] = h

    return pl.pallas_call(
        body,
        grid=grid,
        in_specs=[
            pl.BlockSpec((2, BR, D), lambda i: (0, i, 0)),
            pl.BlockSpec((D, D), lambda i: (0, 0)),
            pl.BlockSpec((D, D), lambda i: (0, 0)),
        ],
        out_specs=pl.BlockSpec((BR, D), lambda i: (i, 0)),
        out_shape=jax.ShapeDtypeStruct((N_NODES, D), jnp.float32),
    )(p, wa, wb)


def _pack_edges(edge_index):
    ei = edge_index.astype(jnp.int32)
    idx = jnp.stack([ei[0].reshape(NW, CHUNKS, K),
                     ei[1].reshape(NW, CHUNKS, K)], axis=2)
    extra = jnp.zeros((NW, 3, 2, K), jnp.int32)
    return jnp.concatenate([idx, extra], axis=1)


def kernel(x, edge_index, w1a, w1b, w2a, w2b):
    idx = _pack_edges(edge_index)
    zeros = jnp.zeros((N_NODES, D), jnp.float32)

    p = _seg_sum_partials(x, idx, zeros)
    h1 = _mlp(p, w1a, w1b, normalize=False)
    q = _seg_sum_partials(h1, idx, zeros)
    return _mlp(q, w2a, w2b, normalize=True)
